# baseline (speedup 1.0000x reference)
import functools

import jax
import jax.numpy as jnp
from jax import lax
from jax.experimental import pallas as pl
from jax.experimental.pallas import tpu as pltpu
from jax.experimental.pallas import tpu_sc as plsc


@functools.lru_cache(maxsize=None)
def _make_gather(dim, batch):
    info = plsc.get_sparse_core_info()
    nc, ns = info.num_cores, info.num_subcores
    nw = nc * ns
    b_per_w = batch // nw
    chunk = 128
    n_chunks = b_per_w // chunk
    mesh = plsc.VectorSubcoreMesh(core_axis_name="c", subcore_axis_name="s")

    @functools.partial(
        pl.kernel,
        mesh=mesh,
        out_type=jax.ShapeDtypeStruct((batch, dim), jnp.float32),
        scratch_types=[
            pltpu.VMEM((chunk,), jnp.int32),
            pltpu.VMEM((chunk, dim), jnp.float32),
            pltpu.SemaphoreType.DMA,
        ],
        compiler_params=pltpu.CompilerParams(use_tc_tiling_on_sc=False),
    )
    def gather_kernel(table_hbm, idx_hbm, out_hbm, idx_v, rows_v, sem):
        wid = lax.axis_index("s") * nc + lax.axis_index("c")
        base = wid * b_per_w

        def body(i, carry):
            off = base + i * chunk
            pltpu.sync_copy(idx_hbm.at[pl.ds(off, chunk)], idx_v)
            pltpu.async_copy(table_hbm.at[idx_v], rows_v, sem).wait()
            pltpu.sync_copy(rows_v, out_hbm.at[pl.ds(off, chunk)])
            return carry

        lax.fori_loop(0, n_chunks, body, 0)

    return gather_kernel


def kernel(token_ids, embedding_matrix):
    b0, b1 = token_ids.shape
    _, d = embedding_matrix.shape
    flat = token_ids.reshape(-1).astype(jnp.int32)
    out = _make_gather(d, b0 * b1)(embedding_matrix, flat)
    return out.reshape(b0, b1, d)


# speedup vs baseline: 1.0764x; 1.0764x over previous
import functools

import jax
import jax.numpy as jnp
from jax import lax
from jax.experimental import pallas as pl
from jax.experimental.pallas import tpu as pltpu
from jax.experimental.pallas import tpu_sc as plsc

_STEP = 640
_NSLOT = 2


@functools.lru_cache(maxsize=None)
def _make_gather(dim, batch):
    info = plsc.get_sparse_core_info()
    nc, ns = info.num_cores, info.num_subcores
    nw = nc * ns
    b_per_w = batch // nw
    n_steps = b_per_w // _STEP
    mesh = plsc.VectorSubcoreMesh(core_axis_name="c", subcore_axis_name="s")

    @functools.partial(
        pl.kernel,
        mesh=mesh,
        out_type=jax.ShapeDtypeStruct((batch, dim), jnp.float32),
        scratch_types=[
            pltpu.VMEM((b_per_w,), jnp.int32),
            pltpu.VMEM((_NSLOT, _STEP, dim), jnp.float32),
            pltpu.SemaphoreType.DMA((_NSLOT,)),
            pltpu.SemaphoreType.DMA((_NSLOT,)),
            pltpu.SemaphoreType.DMA,
        ],
        compiler_params=pltpu.CompilerParams(use_tc_tiling_on_sc=False),
    )
    def gather_kernel(table_hbm, idx_hbm, out_hbm, idx_v, rows_v, gsem, osem, isem):
        wid = lax.axis_index("s") * nc + lax.axis_index("c")
        base = wid * b_per_w
        pltpu.async_copy(idx_hbm.at[pl.ds(base, b_per_w)], idx_v, isem).wait()

        def gather_copy(s):
            b = s % _NSLOT
            return pltpu.make_async_copy(
                table_hbm.at[idx_v.at[pl.ds(s * _STEP, _STEP)]],
                rows_v.at[b],
                gsem.at[b],
            )

        def out_copy(s):
            b = s % _NSLOT
            return pltpu.make_async_copy(
                rows_v.at[b],
                out_hbm.at[pl.ds(base + s * _STEP, _STEP)],
                osem.at[b],
            )

        for s in range(n_steps):
            if s >= _NSLOT:
                out_copy(s - _NSLOT).wait()
            gather_copy(s).start()
            if s >= 1:
                gather_copy(s - 1).wait()
                out_copy(s - 1).start()
        gather_copy(n_steps - 1).wait()
        out_copy(n_steps - 1).start()
        for s in range(max(n_steps - _NSLOT, 0), n_steps):
            out_copy(s).wait()

    return gather_kernel


def kernel(token_ids, embedding_matrix):
    b0, b1 = token_ids.shape
    _, d = embedding_matrix.shape
    batch = b0 * b1
    flat = token_ids.reshape(batch).astype(jnp.int32)
    out = _make_gather(d, batch)(embedding_matrix, flat)
    return out.reshape(b0, b1, d)
